# Initial kernel scaffold; baseline (speedup 1.0000x reference)
import jax
import jax.numpy as jnp
from jax.experimental import pallas as pl


def kernel(n, table):
    raise NotImplementedError("write your pallas kernel here")


# speedup vs baseline: 68.7866x; 68.7866x over previous
import math

import jax
import jax.numpy as jnp
from jax.experimental import pallas as pl
from jax.experimental.pallas import tpu as pltpu

N = 2048
H = 12
NBUCKETS = 32
MAX_DISTANCE = 128
D = 2 * N
BI = 64


def _bias_kernel(tableT_ref, out_ref, rot_ref):
    h = pl.program_id(0)
    ib = pl.program_id(1)

    @pl.when(ib == 0)
    def _build_rotations():
        p = jax.lax.broadcasted_iota(jnp.int32, (1, D), 1)
        d = p - (N - 1)
        nneg = -d
        half = NBUCKETS // 2
        max_exact = half // 2
        ret = jnp.where(nneg < 0, half, 0).astype(jnp.int32)
        na = jnp.abs(nneg)
        is_small = na < max_exact
        naf = jnp.maximum(na, 1).astype(jnp.float32)
        val_large = max_exact + (
            jnp.log(naf / max_exact)
            / math.log(MAX_DISTANCE / max_exact)
            * (half - max_exact)
        ).astype(jnp.int32)
        val_large = jnp.minimum(val_large, half - 1)
        bucket = ret + jnp.where(is_small, na, val_large)

        rows = jax.lax.broadcasted_iota(jnp.int32, (NBUCKETS, D), 0)
        onehot = (jnp.broadcast_to(bucket, (NBUCKETS, D)) == rows).astype(
            jnp.float32
        )
        trow = tableT_ref[pl.ds(h, 1), :]
        vals = jnp.dot(trow, onehot, preferred_element_type=jnp.float32)

        mask = -(((d > 32) | (d < -32)).astype(jnp.float32) * 100000000.0)
        diag = vals + mask

        bc = jnp.broadcast_to(diag, (128, D))
        rot_ref[...] = pltpu.roll(bc, D - 127, 1, stride=1, stride_axis=0)

    i0 = ib * BI
    for r in range(BI):
        w = (N - 1) - (i0 + r)
        b = jax.lax.bitwise_and(w, 127)
        a = pl.multiple_of(w - b, 128)
        row = rot_ref[pl.ds(127 - b, 1), pl.ds(a, N)]
        out_ref[0, r, :] = row[0]


@jax.jit
def _bias_impl(table):
    tableT = table.T
    return pl.pallas_call(
        _bias_kernel,
        grid=(H, N // BI),
        in_specs=[pl.BlockSpec((H, NBUCKETS), lambda h, ib: (0, 0))],
        out_specs=pl.BlockSpec((1, BI, N), lambda h, ib: (h, ib, 0)),
        out_shape=jax.ShapeDtypeStruct((H, N, N), jnp.float32),
        scratch_shapes=[pltpu.VMEM((128, D), jnp.float32)],
        compiler_params=pltpu.CompilerParams(
            dimension_semantics=("arbitrary", "arbitrary"),
        ),
    )(tableT)


def kernel(n, table):
    del n
    return _bias_impl(table)


# speedup vs baseline: 145.1915x; 2.1108x over previous
import math

import jax
import jax.numpy as jnp
from jax.experimental import pallas as pl
from jax.experimental.pallas import tpu as pltpu

N = 2048
H = 12
NBUCKETS = 32
MAX_DISTANCE = 128
D = 2 * N
BI = 128


def _bias_kernel(tableT_ref, out_ref, rot_ref):
    h = pl.program_id(0)
    ib = pl.program_id(1)

    @pl.when(ib == 0)
    def _build_rotations():
        p = jax.lax.broadcasted_iota(jnp.int32, (1, D), 1)
        d = p - (N - 1)
        nneg = -d
        half = NBUCKETS // 2
        max_exact = half // 2
        ret = jnp.where(nneg < 0, half, 0).astype(jnp.int32)
        na = jnp.abs(nneg)
        is_small = na < max_exact
        naf = jnp.maximum(na, 1).astype(jnp.float32)
        val_large = max_exact + (
            jnp.log(naf / max_exact)
            / math.log(MAX_DISTANCE / max_exact)
            * (half - max_exact)
        ).astype(jnp.int32)
        val_large = jnp.minimum(val_large, half - 1)
        bucket = ret + jnp.where(is_small, na, val_large)

        rows = jax.lax.broadcasted_iota(jnp.int32, (NBUCKETS, D), 0)
        onehot = (jnp.broadcast_to(bucket, (NBUCKETS, D)) == rows).astype(
            jnp.float32
        )
        trow = tableT_ref[pl.ds(h, 1), :]
        vals = jnp.dot(trow, onehot, preferred_element_type=jnp.float32)

        mask = -(((d > 32) | (d < -32)).astype(jnp.float32) * 100000000.0)
        diag = vals + mask

        bc = jnp.broadcast_to(diag, (128, D))
        rot_ref[...] = pltpu.roll(bc, D - 127, 1, stride=1, stride_axis=0)

    a = pl.multiple_of((N - 1 - 127) - ib * BI, 128)
    out_ref[0, :, :] = rot_ref[:, pl.ds(a, N)]


@jax.jit
def _bias_impl(table):
    tableT = table.T
    return pl.pallas_call(
        _bias_kernel,
        grid=(H, N // BI),
        in_specs=[pl.BlockSpec((H, NBUCKETS), lambda h, ib: (0, 0))],
        out_specs=pl.BlockSpec((1, BI, N), lambda h, ib: (h, ib, 0)),
        out_shape=jax.ShapeDtypeStruct((H, N, N), jnp.float32),
        scratch_shapes=[pltpu.VMEM((128, D), jnp.float32)],
        compiler_params=pltpu.CompilerParams(
            dimension_semantics=("arbitrary", "arbitrary"),
        ),
    )(tableT)


def kernel(n, table):
    del n
    return _bias_impl(table)


# speedup vs baseline: 200.7344x; 1.3825x over previous
import math

import jax
import jax.numpy as jnp
from jax.experimental import pallas as pl
from jax.experimental.pallas import tpu as pltpu

N = 2048
H = 12
NBUCKETS = 32
MAX_DISTANCE = 128
D = 2 * N
BI = 256


def _bias_kernel(tableT_ref, out_ref, rot_ref):
    h = pl.program_id(0)
    ib = pl.program_id(1)

    @pl.when(ib == 0)
    def _build_rotations():
        p = jax.lax.broadcasted_iota(jnp.int32, (1, D), 1)
        d = p - (N - 1)
        nneg = -d
        half = NBUCKETS // 2
        max_exact = half // 2
        ret = jnp.where(nneg < 0, half, 0).astype(jnp.int32)
        na = jnp.abs(nneg)
        is_small = na < max_exact
        naf = jnp.maximum(na, 1).astype(jnp.float32)
        val_large = max_exact + (
            jnp.log(naf / max_exact)
            / math.log(MAX_DISTANCE / max_exact)
            * (half - max_exact)
        ).astype(jnp.int32)
        val_large = jnp.minimum(val_large, half - 1)
        bucket = ret + jnp.where(is_small, na, val_large)

        rows = jax.lax.broadcasted_iota(jnp.int32, (NBUCKETS, D), 0)
        onehot = (jnp.broadcast_to(bucket, (NBUCKETS, D)) == rows).astype(
            jnp.float32
        )
        trow = tableT_ref[pl.ds(h, 1), :]
        vals = jnp.dot(trow, onehot, preferred_element_type=jnp.float32)

        mask = -(((d > 32) | (d < -32)).astype(jnp.float32) * 100000000.0)
        diag = vals + mask

        bc = jnp.broadcast_to(diag, (128, D))
        rot_ref[...] = pltpu.roll(bc, D - 127, 1, stride=1, stride_axis=0)

    for g in range(BI // 128):
        a = pl.multiple_of((N - 1 - 127) - (ib * BI + g * 128), 128)
        out_ref[0, g * 128:(g + 1) * 128, :] = rot_ref[:, pl.ds(a, N)]


@jax.jit
def _bias_impl(table):
    tableT = table.T
    return pl.pallas_call(
        _bias_kernel,
        grid=(H, N // BI),
        in_specs=[pl.BlockSpec((H, NBUCKETS), lambda h, ib: (0, 0))],
        out_specs=pl.BlockSpec((1, BI, N), lambda h, ib: (h, ib, 0)),
        out_shape=jax.ShapeDtypeStruct((H, N, N), jnp.float32),
        scratch_shapes=[pltpu.VMEM((128, D), jnp.float32)],
        compiler_params=pltpu.CompilerParams(
            dimension_semantics=("arbitrary", "arbitrary"),
        ),
    )(tableT)


def kernel(n, table):
    del n
    return _bias_impl(table)


# speedup vs baseline: 236.8586x; 1.1800x over previous
import math

import jax
import jax.numpy as jnp
from jax.experimental import pallas as pl
from jax.experimental.pallas import tpu as pltpu

N = 2048
H = 12
NBUCKETS = 32
MAX_DISTANCE = 128
D = 2 * N
BI = 512


def _bias_kernel(tableT_ref, out_ref, rot_ref):
    h = pl.program_id(0)
    ib = pl.program_id(1)

    @pl.when(ib == 0)
    def _build_rotations():
        p = jax.lax.broadcasted_iota(jnp.int32, (1, D), 1)
        d = p - (N - 1)
        nneg = -d
        half = NBUCKETS // 2
        max_exact = half // 2
        ret = jnp.where(nneg < 0, half, 0).astype(jnp.int32)
        na = jnp.abs(nneg)
        is_small = na < max_exact
        naf = jnp.maximum(na, 1).astype(jnp.float32)
        val_large = max_exact + (
            jnp.log(naf / max_exact)
            / math.log(MAX_DISTANCE / max_exact)
            * (half - max_exact)
        ).astype(jnp.int32)
        val_large = jnp.minimum(val_large, half - 1)
        bucket = ret + jnp.where(is_small, na, val_large)

        rows = jax.lax.broadcasted_iota(jnp.int32, (NBUCKETS, D), 0)
        onehot = (jnp.broadcast_to(bucket, (NBUCKETS, D)) == rows).astype(
            jnp.float32
        )
        trow = tableT_ref[pl.ds(h, 1), :]
        vals = jnp.dot(trow, onehot, preferred_element_type=jnp.float32)

        mask = -(((d > 32) | (d < -32)).astype(jnp.float32) * 100000000.0)
        diag = vals + mask

        bc = jnp.broadcast_to(diag, (128, D))
        rot_ref[...] = pltpu.roll(bc, D - 127, 1, stride=1, stride_axis=0)

    for g in range(BI // 128):
        a = pl.multiple_of((N - 1 - 127) - (ib * BI + g * 128), 128)
        out_ref[0, g * 128:(g + 1) * 128, :] = rot_ref[:, pl.ds(a, N)]


@jax.jit
def _bias_impl(table):
    tableT = table.T
    return pl.pallas_call(
        _bias_kernel,
        grid=(H, N // BI),
        in_specs=[pl.BlockSpec((H, NBUCKETS), lambda h, ib: (0, 0))],
        out_specs=pl.BlockSpec((1, BI, N), lambda h, ib: (h, ib, 0)),
        out_shape=jax.ShapeDtypeStruct((H, N, N), jnp.float32),
        scratch_shapes=[pltpu.VMEM((128, D), jnp.float32)],
        compiler_params=pltpu.CompilerParams(
            dimension_semantics=("arbitrary", "arbitrary"),
        ),
    )(tableT)


def kernel(n, table):
    del n
    return _bias_impl(table)


# speedup vs baseline: 253.8602x; 1.0718x over previous
import math

import jax
import jax.numpy as jnp
from jax.experimental import pallas as pl
from jax.experimental.pallas import tpu as pltpu

N = 2048
H = 12
NBUCKETS = 32
MAX_DISTANCE = 128
D = 2 * N
BI = 1024


def _bias_kernel(tableT_ref, out_ref, rot_ref):
    h = pl.program_id(0)
    ib = pl.program_id(1)

    @pl.when(ib == 0)
    def _build_rotations():
        p = jax.lax.broadcasted_iota(jnp.int32, (1, D), 1)
        d = p - (N - 1)
        nneg = -d
        half = NBUCKETS // 2
        max_exact = half // 2
        ret = jnp.where(nneg < 0, half, 0).astype(jnp.int32)
        na = jnp.abs(nneg)
        is_small = na < max_exact
        naf = jnp.maximum(na, 1).astype(jnp.float32)
        val_large = max_exact + (
            jnp.log(naf / max_exact)
            / math.log(MAX_DISTANCE / max_exact)
            * (half - max_exact)
        ).astype(jnp.int32)
        val_large = jnp.minimum(val_large, half - 1)
        bucket = ret + jnp.where(is_small, na, val_large)

        rows = jax.lax.broadcasted_iota(jnp.int32, (NBUCKETS, D), 0)
        onehot = (jnp.broadcast_to(bucket, (NBUCKETS, D)) == rows).astype(
            jnp.float32
        )
        trow = tableT_ref[pl.ds(h, 1), :]
        vals = jnp.dot(trow, onehot, preferred_element_type=jnp.float32)

        mask = -(((d > 32) | (d < -32)).astype(jnp.float32) * 100000000.0)
        diag = vals + mask

        bc = jnp.broadcast_to(diag, (128, D))
        rot_ref[...] = pltpu.roll(bc, D - 127, 1, stride=1, stride_axis=0)

    for g in range(BI // 128):
        a = pl.multiple_of((N - 1 - 127) - (ib * BI + g * 128), 128)
        out_ref[0, g * 128:(g + 1) * 128, :] = rot_ref[:, pl.ds(a, N)]


@jax.jit
def _bias_impl(table):
    tableT = table.T
    return pl.pallas_call(
        _bias_kernel,
        grid=(H, N // BI),
        in_specs=[pl.BlockSpec((H, NBUCKETS), lambda h, ib: (0, 0))],
        out_specs=pl.BlockSpec((1, BI, N), lambda h, ib: (h, ib, 0)),
        out_shape=jax.ShapeDtypeStruct((H, N, N), jnp.float32),
        scratch_shapes=[pltpu.VMEM((128, D), jnp.float32)],
        compiler_params=pltpu.CompilerParams(
            dimension_semantics=("arbitrary", "arbitrary"),
        ),
    )(tableT)


def kernel(n, table):
    del n
    return _bias_impl(table)
